# Initial kernel scaffold; baseline (speedup 1.0000x reference)
import jax
import jax.numpy as jnp
from jax.experimental import pallas as pl


def kernel(x, edge_index, edge_attr, W0, b0, Ws, bs, Wes, gammas, betas):
    raise NotImplementedError("write your pallas kernel here")


# speedup vs baseline: 7.4914x; 7.4914x over previous
import functools

import jax
import jax.numpy as jnp
from jax import lax
from jax.experimental import pallas as pl
from jax.experimental.pallas import tpu as pltpu
from jax.experimental.pallas import tpu_sc as plsc

N = 10000
E = 320000
D = 128
D_EDGE = 16
L = 3
EPS = 1e-5

NC = 2
NS = 16
NW = NC * NS
EPT = E // NW
CH = 80
NCHUNK = EPT // CH
NPAD = 10112
RPT = NPAD // NS

_MESH = plsc.VectorSubcoreMesh(
    core_axis_name="c", subcore_axis_name="s", num_cores=NC, num_subcores=NS
)

def _wid():
    return lax.axis_index("c") * NS + lax.axis_index("s")


def _zero_rows(zbuf, table, row0, width_rows):
    zerof = jnp.zeros((16,), jnp.float32)
    for r in range(zbuf.shape[0]):
        for g in range(zbuf.shape[1] // 16):
            zbuf[r, pl.ds(g * 16, 16)] = zerof
    nfull = RPT // 16

    def zloop(i, carry):
        pltpu.sync_copy(zbuf, table.at[pl.ds(row0 + i * 16, 16)])
        return carry

    lax.fori_loop(0, nfull, zloop, 0)
    rem = RPT - nfull * 16
    if rem:
        pltpu.sync_copy(zbuf.at[pl.ds(0, rem)], table.at[pl.ds(row0 + nfull * 16, rem)])


def _load_dst_idx(dst_hbm, dst_v, base):
    def dloop(i, carry):
        pltpu.sync_copy(dst_hbm.at[pl.ds(base + i * CH, CH)], dst_v.at[i])
        return carry

    lax.fori_loop(0, NCHUNK, dloop, 0)


def _load_src_idx(src_hbm, src_v, base):
    pltpu.sync_copy(src_hbm.at[pl.ds(base, EPT)], src_v.at[pl.ds(0, EPT)])


def _copy_out(table, buf, out_hbm, core, row0, width):
    rows_per = buf.shape[0]
    nfull = RPT // rows_per

    def oloop(i, carry):
        r = row0 + i * rows_per
        pltpu.sync_copy(table.at[pl.ds(r, rows_per)], buf)
        pltpu.sync_copy(buf, out_hbm.at[core, pl.ds(r, rows_per)])
        return carry

    lax.fori_loop(0, nfull, oloop, 0)
    rem = RPT - nfull * rows_per
    if rem:
        r = row0 + nfull * rows_per
        pltpu.sync_copy(table.at[pl.ds(r, rem)], buf.at[pl.ds(0, rem)])
        pltpu.sync_copy(buf.at[pl.ds(0, rem)], out_hbm.at[core, pl.ds(r, rem)])


def _deg_body(dst_hbm, out_hbm, dst_v, cbuf, zbuf, deg_sh):
    core = lax.axis_index("c")
    w = _wid()
    base = w * EPT
    row0 = lax.axis_index("s") * RPT

    one0 = jnp.where(
        lax.broadcasted_iota(jnp.int32, (16,), 0) == 0,
        jnp.float32(1.0), jnp.float32(0.0))
    for r in range(CH):
        cbuf[r, :] = one0

    _zero_rows(zbuf, deg_sh, row0, D_EDGE)
    _load_dst_idx(dst_hbm, dst_v, base)
    plsc.subcore_barrier()

    def chunk(i, carry):
        pltpu.sync_copy(cbuf, deg_sh.at[dst_v.at[i]], add=True)
        return carry

    lax.fori_loop(0, NCHUNK, chunk, 0)
    plsc.subcore_barrier()
    _copy_out(deg_sh, cbuf, out_hbm, core, row0, D_EDGE)


_deg_kernel = pl.kernel(
    _deg_body,
    out_type=jax.ShapeDtypeStruct((NC, NPAD, D_EDGE), jnp.float32),
    mesh=_MESH,
    scratch_types=[
        pltpu.VMEM((NCHUNK, CH), jnp.int32),
        pltpu.VMEM((CH, D_EDGE), jnp.float32),
        pltpu.VMEM((16, D_EDGE), jnp.float32),
        pltpu.VMEM_SHARED((NPAD, D_EDGE), jnp.float32),
    ],
)


def _ea_body(ea_hbm, src_hbm, dst_hbm, disc_hbm, out_hbm, src_v, dst_v, ea_buf, wbuf, zbuf, ea_sh):
    core = lax.axis_index("c")
    w = _wid()
    base = w * EPT
    row0 = lax.axis_index("s") * RPT

    _zero_rows(zbuf, ea_sh, row0, D_EDGE)
    _load_src_idx(src_hbm, src_v, base)
    _load_dst_idx(dst_hbm, dst_v, base)
    plsc.subcore_barrier()

    def chunk(i, carry):
        pltpu.sync_copy(ea_hbm.at[pl.ds(base + i * CH, CH)], ea_buf)
        pltpu.sync_copy(disc_hbm.at[src_v.at[pl.ds(i * CH, CH)]], wbuf)
        for r in range(CH):
            ea_buf[r, :] = ea_buf[r, :] * wbuf[r, pl.ds(0, D_EDGE)]
        pltpu.sync_copy(ea_buf, ea_sh.at[dst_v.at[i]], add=True)
        return carry

    lax.fori_loop(0, NCHUNK, chunk, 0)
    plsc.subcore_barrier()
    _copy_out(ea_sh, ea_buf, out_hbm, core, row0, D_EDGE)


_ea_kernel = pl.kernel(
    _ea_body,
    out_type=jax.ShapeDtypeStruct((NC, NPAD, D_EDGE), jnp.float32),
    mesh=_MESH,
    scratch_types=[
        pltpu.VMEM((EPT,), jnp.int32),
        pltpu.VMEM((NCHUNK, CH), jnp.int32),
        pltpu.VMEM((CH, D_EDGE), jnp.float32),
        pltpu.VMEM((CH, D), jnp.float32),
        pltpu.VMEM((16, D_EDGE), jnp.float32),
        pltpu.VMEM_SHARED((NPAD, D_EDGE), jnp.float32),
    ],
)


def _spmm_body(hs_hbm, src_hbm, dst_hbm, out_hbm, src_v, dst_v, buf, zbuf, agg_sh):
    core = lax.axis_index("c")
    w = _wid()
    base = w * EPT
    row0 = lax.axis_index("s") * RPT

    _zero_rows(zbuf, agg_sh, row0, D)
    _load_src_idx(src_hbm, src_v, base)
    _load_dst_idx(dst_hbm, dst_v, base)
    plsc.subcore_barrier()

    def chunk(i, carry):
        pltpu.sync_copy(hs_hbm.at[src_v.at[pl.ds(i * CH, CH)]], buf)
        pltpu.sync_copy(buf, agg_sh.at[dst_v.at[i]], add=True)
        return carry

    lax.fori_loop(0, NCHUNK, chunk, 0)
    plsc.subcore_barrier()
    _copy_out(agg_sh, buf, out_hbm, core, row0, D)


_spmm_kernel = pl.kernel(
    _spmm_body,
    out_type=jax.ShapeDtypeStruct((NC, NPAD, D), jnp.float32),
    mesh=_MESH,
    scratch_types=[
        pltpu.VMEM((EPT,), jnp.int32),
        pltpu.VMEM((NCHUNK, CH), jnp.int32),
        pltpu.VMEM((CH, D), jnp.float32),
        pltpu.VMEM((16, D), jnp.float32),
        pltpu.VMEM_SHARED((NPAD, D), jnp.float32),
    ],
)


RB = 1000


def _tc_b_body(x_ref, w0_ref, b0_ref, ws_ref, bs_ref, degp_ref, hs_ref, dis_ref):
    deg = degp_ref[0, :, 0] + degp_ref[1, :, 0] + 1.0
    dis = lax.rsqrt(deg)[:, None]
    h = jnp.maximum(
        jnp.dot(x_ref[...], w0_ref[...], preferred_element_type=jnp.float32)
        + b0_ref[...], 0.0)
    hl = jnp.dot(h, ws_ref[...], preferred_element_type=jnp.float32) + bs_ref[...]
    hs_ref[...] = hl * dis
    dis_ref[...] = jnp.broadcast_to(dis, (RB, D))


_tc_b = pl.pallas_call(
    _tc_b_body,
    grid=(N // RB,),
    in_specs=[
        pl.BlockSpec((RB, D), lambda i: (i, 0)),
        pl.BlockSpec((D, D), lambda i: (0, 0)),
        pl.BlockSpec((1, D), lambda i: (0, 0)),
        pl.BlockSpec((D, D), lambda i: (0, 0)),
        pl.BlockSpec((1, D), lambda i: (0, 0)),
        pl.BlockSpec((NC, RB, D_EDGE), lambda i: (0, i, 0)),
    ],
    out_specs=[
        pl.BlockSpec((RB, D), lambda i: (i, 0)),
        pl.BlockSpec((RB, D), lambda i: (i, 0)),
    ],
    out_shape=[
        jax.ShapeDtypeStruct((N, D), jnp.float32),
        jax.ShapeDtypeStruct((N, D), jnp.float32),
    ],
)


def _tc_e_body(p_ref, hs_ref, dis_ref, eap_ref, wes_ref, sg_ref, beta_ref,
               wn_ref, bn_ref, out_ref, *, last):
    s = p_ref[0] + p_ref[1]
    ea = eap_ref[0] + eap_ref[1]
    dis = dis_ref[...]
    agg = dis * (
        s + jnp.dot(ea, wes_ref[...], preferred_element_type=jnp.float32)
        + hs_ref[...])
    hbn = agg * sg_ref[...] + beta_ref[...]
    if last:
        out_ref[...] = hbn
    else:
        h = jnp.maximum(hbn, 0.0)
        out_ref[...] = (
            jnp.dot(h, wn_ref[...], preferred_element_type=jnp.float32)
            + bn_ref[...]) * dis


def _make_tc_e(last):
    return pl.pallas_call(
        functools.partial(_tc_e_body, last=last),
        grid=(N // RB,),
        in_specs=[
            pl.BlockSpec((NC, RB, D), lambda i: (0, i, 0)),
            pl.BlockSpec((RB, D), lambda i: (i, 0)),
            pl.BlockSpec((RB, D), lambda i: (i, 0)),
            pl.BlockSpec((NC, RB, D_EDGE), lambda i: (0, i, 0)),
            pl.BlockSpec((D_EDGE, D), lambda i: (0, 0)),
            pl.BlockSpec((1, D), lambda i: (0, 0)),
            pl.BlockSpec((1, D), lambda i: (0, 0)),
            pl.BlockSpec((D, D), lambda i: (0, 0)),
            pl.BlockSpec((1, D), lambda i: (0, 0)),
        ],
        out_specs=pl.BlockSpec((RB, D), lambda i: (i, 0)),
        out_shape=jax.ShapeDtypeStruct((N, D), jnp.float32),
    )


_tc_e_mid = _make_tc_e(last=False)
_tc_e_last = _make_tc_e(last=True)


def kernel(x, edge_index, edge_attr, W0, b0, Ws, bs, Wes, gammas, betas):
    ei = edge_index.astype(jnp.int32)
    src_e, dst_e = ei[0], ei[1]
    sg = (gammas / jnp.sqrt(1.0 + EPS)).astype(jnp.float32)

    degp = _deg_kernel(dst_e)
    hs, dis_col = _tc_b(
        x, W0, b0.reshape(1, D), Ws[0], bs[0].reshape(1, D), degp)
    eap = _ea_kernel(edge_attr, src_e, dst_e, dis_col)

    for l in range(L):
        part = _spmm_kernel(hs, src_e, dst_e)
        if l < L - 1:
            hs = _tc_e_mid(
                part, hs, dis_col, eap, Wes[l], sg[l].reshape(1, D),
                betas[l].reshape(1, D), Ws[l + 1], bs[l + 1].reshape(1, D))
        else:
            out = _tc_e_last(
                part, hs, dis_col, eap, Wes[l], sg[l].reshape(1, D),
                betas[l].reshape(1, D), Ws[l], bs[l].reshape(1, D))
    return out


# speedup vs baseline: 7.5187x; 1.0036x over previous
import functools

import jax
import jax.numpy as jnp
from jax import lax
from jax.experimental import pallas as pl
from jax.experimental.pallas import tpu as pltpu
from jax.experimental.pallas import tpu_sc as plsc

N = 10000
E = 320000
D = 128
D_EDGE = 16
L = 3
EPS = 1e-5

NC = 2
NS = 16
NW = NC * NS
EPT = E // NW
CH = 80
NCHUNK = EPT // CH
NPAD = 10112
RPT = NPAD // NS

_MESH = plsc.VectorSubcoreMesh(
    core_axis_name="c", subcore_axis_name="s", num_cores=NC, num_subcores=NS
)

def _wid():
    return lax.axis_index("c") * NS + lax.axis_index("s")


def _zero_rows(zbuf, table, row0, width_rows):
    zerof = jnp.zeros((16,), jnp.float32)
    for r in range(zbuf.shape[0]):
        for g in range(zbuf.shape[1] // 16):
            zbuf[r, pl.ds(g * 16, 16)] = zerof
    rows = zbuf.shape[0]
    nfull = RPT // rows

    def zloop(i, carry):
        pltpu.sync_copy(zbuf, table.at[pl.ds(row0 + i * rows, rows)])
        return carry

    lax.fori_loop(0, nfull, zloop, 0)
    rem = RPT - nfull * rows
    if rem:
        pltpu.sync_copy(zbuf.at[pl.ds(0, rem)],
                        table.at[pl.ds(row0 + nfull * rows, rem)])


def _load_dst_idx(dst_hbm, dst_v, base):
    def dloop(i, carry):
        pltpu.sync_copy(dst_hbm.at[pl.ds(base + i * CH, CH)], dst_v.at[i])
        return carry

    lax.fori_loop(0, NCHUNK, dloop, 0)


def _load_src_idx(src_hbm, src_v, base):
    pltpu.sync_copy(src_hbm.at[pl.ds(base, EPT)], src_v.at[pl.ds(0, EPT)])


def _copy_out(table, buf, out_hbm, core, row0, width):
    rows_per = buf.shape[0]
    nfull = RPT // rows_per

    def oloop(i, carry):
        r = row0 + i * rows_per
        pltpu.sync_copy(table.at[pl.ds(r, rows_per)], buf)
        pltpu.sync_copy(buf, out_hbm.at[core, pl.ds(r, rows_per)])
        return carry

    lax.fori_loop(0, nfull, oloop, 0)
    rem = RPT - nfull * rows_per
    if rem:
        r = row0 + nfull * rows_per
        pltpu.sync_copy(table.at[pl.ds(r, rem)], buf.at[pl.ds(0, rem)])
        pltpu.sync_copy(buf.at[pl.ds(0, rem)], out_hbm.at[core, pl.ds(r, rem)])


def _deg_body(dst_hbm, out_hbm, dst_v, cbuf, deg_sh):
    core = lax.axis_index("c")
    w = _wid()
    base = w * EPT
    row0 = lax.axis_index("s") * RPT

    _zero_rows(cbuf, deg_sh, row0, D_EDGE)
    one0 = jnp.where(
        lax.broadcasted_iota(jnp.int32, (16,), 0) == 0,
        jnp.float32(1.0), jnp.float32(0.0))
    for r in range(CH):
        cbuf[r, :] = one0

    _load_dst_idx(dst_hbm, dst_v, base)
    plsc.subcore_barrier()

    def chunk(i, carry):
        pltpu.sync_copy(cbuf, deg_sh.at[dst_v.at[i]], add=True)
        return carry

    lax.fori_loop(0, NCHUNK, chunk, 0)
    plsc.subcore_barrier()
    _copy_out(deg_sh, cbuf, out_hbm, core, row0, D_EDGE)


_deg_kernel = pl.kernel(
    _deg_body,
    out_type=jax.ShapeDtypeStruct((NC, NPAD, D_EDGE), jnp.float32),
    mesh=_MESH,
    scratch_types=[
        pltpu.VMEM((NCHUNK, CH), jnp.int32),
        pltpu.VMEM((CH, D_EDGE), jnp.float32),
        pltpu.VMEM_SHARED((NPAD, D_EDGE), jnp.float32),
    ],
)


def _ea_body(ea_hbm, src_hbm, dst_hbm, disc_hbm, out_hbm, src_v, dst_v, ea_buf, wbuf, ea_sh):
    core = lax.axis_index("c")
    w = _wid()
    base = w * EPT
    row0 = lax.axis_index("s") * RPT

    _zero_rows(ea_buf, ea_sh, row0, D_EDGE)
    _load_src_idx(src_hbm, src_v, base)
    _load_dst_idx(dst_hbm, dst_v, base)
    plsc.subcore_barrier()

    def chunk(i, carry):
        pltpu.sync_copy(ea_hbm.at[pl.ds(base + i * CH, CH)], ea_buf)
        pltpu.sync_copy(disc_hbm.at[src_v.at[pl.ds(i * CH, CH)]], wbuf)
        for r in range(CH):
            ea_buf[r, :] = ea_buf[r, :] * wbuf[r, pl.ds(0, D_EDGE)]
        pltpu.sync_copy(ea_buf, ea_sh.at[dst_v.at[i]], add=True)
        return carry

    lax.fori_loop(0, NCHUNK, chunk, 0)
    plsc.subcore_barrier()
    _copy_out(ea_sh, ea_buf, out_hbm, core, row0, D_EDGE)


_ea_kernel = pl.kernel(
    _ea_body,
    out_type=jax.ShapeDtypeStruct((NC, NPAD, D_EDGE), jnp.float32),
    mesh=_MESH,
    scratch_types=[
        pltpu.VMEM((EPT,), jnp.int32),
        pltpu.VMEM((NCHUNK, CH), jnp.int32),
        pltpu.VMEM((CH, D_EDGE), jnp.float32),
        pltpu.VMEM((CH, D), jnp.float32),
        pltpu.VMEM_SHARED((NPAD, D_EDGE), jnp.float32),
    ],
)


def _spmm_body(hs_hbm, src_hbm, dst_hbm, out_hbm, src_v, dst_v, buf, agg_sh):
    core = lax.axis_index("c")
    w = _wid()
    base = w * EPT
    row0 = lax.axis_index("s") * RPT

    _zero_rows(buf, agg_sh, row0, D)
    _load_src_idx(src_hbm, src_v, base)
    _load_dst_idx(dst_hbm, dst_v, base)
    plsc.subcore_barrier()

    def chunk(i, carry):
        pltpu.sync_copy(hs_hbm.at[src_v.at[pl.ds(i * CH, CH)]], buf)
        pltpu.sync_copy(buf, agg_sh.at[dst_v.at[i]], add=True)
        return carry

    lax.fori_loop(0, NCHUNK, chunk, 0)
    plsc.subcore_barrier()
    _copy_out(agg_sh, buf, out_hbm, core, row0, D)


_spmm_kernel = pl.kernel(
    _spmm_body,
    out_type=jax.ShapeDtypeStruct((NC, NPAD, D), jnp.float32),
    mesh=_MESH,
    scratch_types=[
        pltpu.VMEM((EPT,), jnp.int32),
        pltpu.VMEM((NCHUNK, CH), jnp.int32),
        pltpu.VMEM((CH, D), jnp.float32),
        pltpu.VMEM_SHARED((NPAD, D), jnp.float32),
    ],
)


RB = 1000


def _tc_b_body(x_ref, w0_ref, b0_ref, ws_ref, bs_ref, degp_ref, hs_ref, dis_ref):
    deg = degp_ref[0, :, 0] + degp_ref[1, :, 0] + 1.0
    dis = lax.rsqrt(deg)[:, None]
    h = jnp.maximum(
        jnp.dot(x_ref[...], w0_ref[...], preferred_element_type=jnp.float32)
        + b0_ref[...], 0.0)
    hl = jnp.dot(h, ws_ref[...], preferred_element_type=jnp.float32) + bs_ref[...]
    hs_ref[...] = hl * dis
    dis_ref[...] = jnp.broadcast_to(dis, (RB, D))


_tc_b = pl.pallas_call(
    _tc_b_body,
    grid=(N // RB,),
    in_specs=[
        pl.BlockSpec((RB, D), lambda i: (i, 0)),
        pl.BlockSpec((D, D), lambda i: (0, 0)),
        pl.BlockSpec((1, D), lambda i: (0, 0)),
        pl.BlockSpec((D, D), lambda i: (0, 0)),
        pl.BlockSpec((1, D), lambda i: (0, 0)),
        pl.BlockSpec((NC, RB, D_EDGE), lambda i: (0, i, 0)),
    ],
    out_specs=[
        pl.BlockSpec((RB, D), lambda i: (i, 0)),
        pl.BlockSpec((RB, D), lambda i: (i, 0)),
    ],
    out_shape=[
        jax.ShapeDtypeStruct((N, D), jnp.float32),
        jax.ShapeDtypeStruct((N, D), jnp.float32),
    ],
)


def _tc_e_body(p_ref, hs_ref, dis_ref, eap_ref, wes_ref, sg_ref, beta_ref,
               wn_ref, bn_ref, out_ref, *, last):
    s = p_ref[0] + p_ref[1]
    ea = eap_ref[0] + eap_ref[1]
    dis = dis_ref[...]
    agg = dis * (
        s + jnp.dot(ea, wes_ref[...], preferred_element_type=jnp.float32)
        + hs_ref[...])
    hbn = agg * sg_ref[...] + beta_ref[...]
    if last:
        out_ref[...] = hbn
    else:
        h = jnp.maximum(hbn, 0.0)
        out_ref[...] = (
            jnp.dot(h, wn_ref[...], preferred_element_type=jnp.float32)
            + bn_ref[...]) * dis


def _make_tc_e(last):
    return pl.pallas_call(
        functools.partial(_tc_e_body, last=last),
        grid=(N // RB,),
        in_specs=[
            pl.BlockSpec((NC, RB, D), lambda i: (0, i, 0)),
            pl.BlockSpec((RB, D), lambda i: (i, 0)),
            pl.BlockSpec((RB, D), lambda i: (i, 0)),
            pl.BlockSpec((NC, RB, D_EDGE), lambda i: (0, i, 0)),
            pl.BlockSpec((D_EDGE, D), lambda i: (0, 0)),
            pl.BlockSpec((1, D), lambda i: (0, 0)),
            pl.BlockSpec((1, D), lambda i: (0, 0)),
            pl.BlockSpec((D, D), lambda i: (0, 0)),
            pl.BlockSpec((1, D), lambda i: (0, 0)),
        ],
        out_specs=pl.BlockSpec((RB, D), lambda i: (i, 0)),
        out_shape=jax.ShapeDtypeStruct((N, D), jnp.float32),
    )


_tc_e_mid = _make_tc_e(last=False)
_tc_e_last = _make_tc_e(last=True)


def kernel(x, edge_index, edge_attr, W0, b0, Ws, bs, Wes, gammas, betas):
    ei = edge_index.astype(jnp.int32)
    src_e, dst_e = ei[0], ei[1]
    sg = (gammas / jnp.sqrt(1.0 + EPS)).astype(jnp.float32)

    degp = _deg_kernel(dst_e)
    hs, dis_col = _tc_b(
        x, W0, b0.reshape(1, D), Ws[0], bs[0].reshape(1, D), degp)
    eap = _ea_kernel(edge_attr, src_e, dst_e, dis_col)

    for l in range(L):
        part = _spmm_kernel(hs, src_e, dst_e)
        if l < L - 1:
            hs = _tc_e_mid(
                part, hs, dis_col, eap, Wes[l], sg[l].reshape(1, D),
                betas[l].reshape(1, D), Ws[l + 1], bs[l + 1].reshape(1, D))
        else:
            out = _tc_e_last(
                part, hs, dis_col, eap, Wes[l], sg[l].reshape(1, D),
                betas[l].reshape(1, D), Ws[l], bs[l].reshape(1, D))
    return out


# speedup vs baseline: 8.9905x; 1.1958x over previous
import functools

import jax
import jax.numpy as jnp
from jax import lax
from jax.experimental import pallas as pl
from jax.experimental.pallas import tpu as pltpu
from jax.experimental.pallas import tpu_sc as plsc

N = 10000
E = 320000
D = 128
D_EDGE = 16
L = 3
EPS = 1e-5

NC = 2
NS = 16
NW = NC * NS
EPT = E // NW
CH = 80
NCHUNK = EPT // CH
NPAD = 10112
RPT = NPAD // NS

_MESH = plsc.VectorSubcoreMesh(
    core_axis_name="c", subcore_axis_name="s", num_cores=NC, num_subcores=NS
)

def _wid():
    return lax.axis_index("c") * NS + lax.axis_index("s")


def _zero_rows(zbuf, table, row0, width_rows):
    zerof = jnp.zeros((16,), jnp.float32)
    for r in range(zbuf.shape[0]):
        for g in range(zbuf.shape[1] // 16):
            zbuf[r, pl.ds(g * 16, 16)] = zerof
    rows = zbuf.shape[0]
    nfull = RPT // rows

    def zloop(i, carry):
        pltpu.sync_copy(zbuf, table.at[pl.ds(row0 + i * rows, rows)])
        return carry

    lax.fori_loop(0, nfull, zloop, 0)
    rem = RPT - nfull * rows
    if rem:
        pltpu.sync_copy(zbuf.at[pl.ds(0, rem)],
                        table.at[pl.ds(row0 + nfull * rows, rem)])


def _load_dst_idx(dst3_hbm, dst_v, w):
    pltpu.sync_copy(dst3_hbm.at[w], dst_v)


def _load_src_idx(src_hbm, src_v, base):
    pltpu.sync_copy(src_hbm.at[pl.ds(base, EPT)], src_v.at[pl.ds(0, EPT)])


def _copy_out(table, buf, out_hbm, core, row0, width):
    rows_per = buf.shape[0]
    nfull = RPT // rows_per

    def oloop(i, carry):
        r = row0 + i * rows_per
        pltpu.sync_copy(table.at[pl.ds(r, rows_per)], buf)
        pltpu.sync_copy(buf, out_hbm.at[core, pl.ds(r, rows_per)])
        return carry

    lax.fori_loop(0, nfull, oloop, 0)
    rem = RPT - nfull * rows_per
    if rem:
        r = row0 + nfull * rows_per
        pltpu.sync_copy(table.at[pl.ds(r, rem)], buf.at[pl.ds(0, rem)])
        pltpu.sync_copy(buf.at[pl.ds(0, rem)], out_hbm.at[core, pl.ds(r, rem)])


def _deg_body(dst3_hbm, out_hbm, dst_v, cbuf, deg_sh):
    core = lax.axis_index("c")
    w = _wid()
    base = w * EPT
    row0 = lax.axis_index("s") * RPT

    _zero_rows(cbuf, deg_sh, row0, D_EDGE)
    one0 = jnp.where(
        lax.broadcasted_iota(jnp.int32, (16,), 0) == 0,
        jnp.float32(1.0), jnp.float32(0.0))
    for r in range(CH):
        cbuf[r, :] = one0

    _load_dst_idx(dst3_hbm, dst_v, w)
    plsc.subcore_barrier()

    def chunk(i, carry):
        pltpu.sync_copy(cbuf, deg_sh.at[dst_v.at[i]], add=True)
        return carry

    lax.fori_loop(0, NCHUNK, chunk, 0)
    plsc.subcore_barrier()
    _copy_out(deg_sh, cbuf, out_hbm, core, row0, D_EDGE)


_deg_kernel = pl.kernel(
    _deg_body,
    out_type=jax.ShapeDtypeStruct((NC, NPAD, D_EDGE), jnp.float32),
    mesh=_MESH,
    scratch_types=[
        pltpu.VMEM((NCHUNK, CH), jnp.int32),
        pltpu.VMEM((CH, D_EDGE), jnp.float32),
        pltpu.VMEM_SHARED((NPAD, D_EDGE), jnp.float32),
    ],
)


def _ea_body(ea_hbm, src_hbm, dst3_hbm, disc_hbm, out_hbm, src_v, dst_v, ea_buf, wbuf, ea_sh):
    core = lax.axis_index("c")
    w = _wid()
    base = w * EPT
    row0 = lax.axis_index("s") * RPT

    _zero_rows(ea_buf, ea_sh, row0, D_EDGE)
    _load_src_idx(src_hbm, src_v, base)
    _load_dst_idx(dst3_hbm, dst_v, w)
    plsc.subcore_barrier()

    def chunk(i, carry):
        pltpu.sync_copy(ea_hbm.at[pl.ds(base + i * CH, CH)], ea_buf)
        pltpu.sync_copy(disc_hbm.at[src_v.at[pl.ds(i * CH, CH)]], wbuf)
        for r in range(CH):
            ea_buf[r, :] = ea_buf[r, :] * wbuf[r, pl.ds(0, D_EDGE)]
        pltpu.sync_copy(ea_buf, ea_sh.at[dst_v.at[i]], add=True)
        return carry

    lax.fori_loop(0, NCHUNK, chunk, 0)
    plsc.subcore_barrier()
    _copy_out(ea_sh, ea_buf, out_hbm, core, row0, D_EDGE)


_ea_kernel = pl.kernel(
    _ea_body,
    out_type=jax.ShapeDtypeStruct((NC, NPAD, D_EDGE), jnp.float32),
    mesh=_MESH,
    scratch_types=[
        pltpu.VMEM((EPT,), jnp.int32),
        pltpu.VMEM((NCHUNK, CH), jnp.int32),
        pltpu.VMEM((CH, D_EDGE), jnp.float32),
        pltpu.VMEM((CH, D), jnp.float32),
        pltpu.VMEM_SHARED((NPAD, D_EDGE), jnp.float32),
    ],
)


def _spmm_body(hs_hbm, src_hbm, dst3_hbm, out_hbm, src_v, dst_v, buf, agg_sh):
    core = lax.axis_index("c")
    w = _wid()
    base = w * EPT
    row0 = lax.axis_index("s") * RPT

    _zero_rows(buf, agg_sh, row0, D)
    _load_src_idx(src_hbm, src_v, base)
    _load_dst_idx(dst3_hbm, dst_v, w)
    plsc.subcore_barrier()

    def chunk(i, carry):
        pltpu.sync_copy(hs_hbm.at[src_v.at[pl.ds(i * CH, CH)]], buf)
        pltpu.sync_copy(buf, agg_sh.at[dst_v.at[i]], add=True)
        return carry

    lax.fori_loop(0, NCHUNK, chunk, 0)
    plsc.subcore_barrier()
    _copy_out(agg_sh, buf, out_hbm, core, row0, D)


_spmm_kernel = pl.kernel(
    _spmm_body,
    out_type=jax.ShapeDtypeStruct((NC, NPAD, D), jnp.float32),
    mesh=_MESH,
    scratch_types=[
        pltpu.VMEM((EPT,), jnp.int32),
        pltpu.VMEM((NCHUNK, CH), jnp.int32),
        pltpu.VMEM((CH, D), jnp.float32),
        pltpu.VMEM_SHARED((NPAD, D), jnp.float32),
    ],
)


RB = 1000


def _tc_b_body(x_ref, w0_ref, b0_ref, ws_ref, bs_ref, degp_ref, hs_ref, dis_ref):
    deg = degp_ref[0, :, 0] + degp_ref[1, :, 0] + 1.0
    dis = lax.rsqrt(deg)[:, None]
    h = jnp.maximum(
        jnp.dot(x_ref[...], w0_ref[...], preferred_element_type=jnp.float32)
        + b0_ref[...], 0.0)
    hl = jnp.dot(h, ws_ref[...], preferred_element_type=jnp.float32) + bs_ref[...]
    hs_ref[...] = hl * dis
    dis_ref[...] = jnp.broadcast_to(dis, (RB, D))


_tc_b = pl.pallas_call(
    _tc_b_body,
    grid=(N // RB,),
    in_specs=[
        pl.BlockSpec((RB, D), lambda i: (i, 0)),
        pl.BlockSpec((D, D), lambda i: (0, 0)),
        pl.BlockSpec((1, D), lambda i: (0, 0)),
        pl.BlockSpec((D, D), lambda i: (0, 0)),
        pl.BlockSpec((1, D), lambda i: (0, 0)),
        pl.BlockSpec((NC, RB, D_EDGE), lambda i: (0, i, 0)),
    ],
    out_specs=[
        pl.BlockSpec((RB, D), lambda i: (i, 0)),
        pl.BlockSpec((RB, D), lambda i: (i, 0)),
    ],
    out_shape=[
        jax.ShapeDtypeStruct((N, D), jnp.float32),
        jax.ShapeDtypeStruct((N, D), jnp.float32),
    ],
)


def _tc_e_body(p_ref, hs_ref, dis_ref, eap_ref, wes_ref, sg_ref, beta_ref,
               wn_ref, bn_ref, out_ref, *, last):
    s = p_ref[0] + p_ref[1]
    ea = eap_ref[0] + eap_ref[1]
    dis = dis_ref[...]
    agg = dis * (
        s + jnp.dot(ea, wes_ref[...], preferred_element_type=jnp.float32)
        + hs_ref[...])
    hbn = agg * sg_ref[...] + beta_ref[...]
    if last:
        out_ref[...] = hbn
    else:
        h = jnp.maximum(hbn, 0.0)
        out_ref[...] = (
            jnp.dot(h, wn_ref[...], preferred_element_type=jnp.float32)
            + bn_ref[...]) * dis


def _make_tc_e(last):
    return pl.pallas_call(
        functools.partial(_tc_e_body, last=last),
        grid=(N // RB,),
        in_specs=[
            pl.BlockSpec((NC, RB, D), lambda i: (0, i, 0)),
            pl.BlockSpec((RB, D), lambda i: (i, 0)),
            pl.BlockSpec((RB, D), lambda i: (i, 0)),
            pl.BlockSpec((NC, RB, D_EDGE), lambda i: (0, i, 0)),
            pl.BlockSpec((D_EDGE, D), lambda i: (0, 0)),
            pl.BlockSpec((1, D), lambda i: (0, 0)),
            pl.BlockSpec((1, D), lambda i: (0, 0)),
            pl.BlockSpec((D, D), lambda i: (0, 0)),
            pl.BlockSpec((1, D), lambda i: (0, 0)),
        ],
        out_specs=pl.BlockSpec((RB, D), lambda i: (i, 0)),
        out_shape=jax.ShapeDtypeStruct((N, D), jnp.float32),
    )


_tc_e_mid = _make_tc_e(last=False)
_tc_e_last = _make_tc_e(last=True)


def kernel(x, edge_index, edge_attr, W0, b0, Ws, bs, Wes, gammas, betas):
    ei = edge_index.astype(jnp.int32)
    src_e, dst_e = ei[0], ei[1]
    sg = (gammas / jnp.sqrt(1.0 + EPS)).astype(jnp.float32)

    dst3 = dst_e.reshape(NW, EPT // CH, CH)
    degp = _deg_kernel(dst3)
    hs, dis_col = _tc_b(
        x, W0, b0.reshape(1, D), Ws[0], bs[0].reshape(1, D), degp)
    eap = _ea_kernel(edge_attr, src_e, dst3, dis_col)

    for l in range(L):
        part = _spmm_kernel(hs, src_e, dst3)
        if l < L - 1:
            hs = _tc_e_mid(
                part, hs, dis_col, eap, Wes[l], sg[l].reshape(1, D),
                betas[l].reshape(1, D), Ws[l + 1], bs[l + 1].reshape(1, D))
        else:
            out = _tc_e_last(
                part, hs, dis_col, eap, Wes[l], sg[l].reshape(1, D),
                betas[l].reshape(1, D), Ws[l], bs[l].reshape(1, D))
    return out


# speedup vs baseline: 9.0163x; 1.0029x over previous
import functools

import jax
import jax.numpy as jnp
from jax import lax
from jax.experimental import pallas as pl
from jax.experimental.pallas import tpu as pltpu
from jax.experimental.pallas import tpu_sc as plsc

N = 10000
E = 320000
D = 128
D_EDGE = 16
L = 3
EPS = 1e-5

NC = 2
NS = 16
NW = NC * NS
EPT = E // NW
CH = 80
NCHUNK = EPT // CH
NPAD = 10112
RPT = NPAD // NS

_MESH = plsc.VectorSubcoreMesh(
    core_axis_name="c", subcore_axis_name="s", num_cores=NC, num_subcores=NS
)

def _wid():
    return lax.axis_index("c") * NS + lax.axis_index("s")


def _zero_rows(zbuf, table, row0, width_rows):
    zerof = jnp.zeros((16,), jnp.float32)
    for r in range(zbuf.shape[0]):
        for g in range(zbuf.shape[1] // 16):
            zbuf[r, pl.ds(g * 16, 16)] = zerof
    rows = zbuf.shape[0]
    nfull = RPT // rows

    def zloop(i, carry):
        pltpu.sync_copy(zbuf, table.at[pl.ds(row0 + i * rows, rows)])
        return carry

    lax.fori_loop(0, nfull, zloop, 0)
    rem = RPT - nfull * rows
    if rem:
        pltpu.sync_copy(zbuf.at[pl.ds(0, rem)],
                        table.at[pl.ds(row0 + nfull * rows, rem)])


def _load_dst_idx(dst3_hbm, dst_v, w):
    pltpu.sync_copy(dst3_hbm.at[w], dst_v)


def _load_src_idx(src_hbm, src_v, base):
    pltpu.sync_copy(src_hbm.at[pl.ds(base, EPT)], src_v.at[pl.ds(0, EPT)])


def _copy_out(table, buf, out_hbm, core, row0, width):
    del buf, width
    pltpu.sync_copy(table.at[pl.ds(row0, RPT)],
                    out_hbm.at[core, pl.ds(row0, RPT)])


def _deg_body(dst3_hbm, out_hbm, dst_v, cbuf, deg_sh):
    core = lax.axis_index("c")
    w = _wid()
    base = w * EPT
    row0 = lax.axis_index("s") * RPT

    _zero_rows(cbuf, deg_sh, row0, D_EDGE)
    one0 = jnp.where(
        lax.broadcasted_iota(jnp.int32, (16,), 0) == 0,
        jnp.float32(1.0), jnp.float32(0.0))
    for r in range(CH):
        cbuf[r, :] = one0

    _load_dst_idx(dst3_hbm, dst_v, w)
    plsc.subcore_barrier()

    def chunk(i, carry):
        pltpu.sync_copy(cbuf, deg_sh.at[dst_v.at[i]], add=True)
        return carry

    lax.fori_loop(0, NCHUNK, chunk, 0)
    plsc.subcore_barrier()
    _copy_out(deg_sh, cbuf, out_hbm, core, row0, D_EDGE)


_deg_kernel = pl.kernel(
    _deg_body,
    out_type=jax.ShapeDtypeStruct((NC, NPAD, D_EDGE), jnp.float32),
    mesh=_MESH,
    scratch_types=[
        pltpu.VMEM((NCHUNK, CH), jnp.int32),
        pltpu.VMEM((CH, D_EDGE), jnp.float32),
        pltpu.VMEM_SHARED((NPAD, D_EDGE), jnp.float32),
    ],
)


def _ea_body(ea_hbm, src_hbm, dst3_hbm, disc_hbm, out_hbm, src_v, dst_v, ea_buf, wbuf, ea_sh):
    core = lax.axis_index("c")
    w = _wid()
    base = w * EPT
    row0 = lax.axis_index("s") * RPT

    _zero_rows(ea_buf, ea_sh, row0, D_EDGE)
    _load_src_idx(src_hbm, src_v, base)
    _load_dst_idx(dst3_hbm, dst_v, w)
    plsc.subcore_barrier()

    def chunk(i, carry):
        pltpu.sync_copy(ea_hbm.at[pl.ds(base + i * CH, CH)], ea_buf)
        pltpu.sync_copy(disc_hbm.at[src_v.at[pl.ds(i * CH, CH)]], wbuf)
        for r in range(CH):
            ea_buf[r, :] = ea_buf[r, :] * wbuf[r, pl.ds(0, D_EDGE)]
        pltpu.sync_copy(ea_buf, ea_sh.at[dst_v.at[i]], add=True)
        return carry

    lax.fori_loop(0, NCHUNK, chunk, 0)
    plsc.subcore_barrier()
    _copy_out(ea_sh, ea_buf, out_hbm, core, row0, D_EDGE)


_ea_kernel = pl.kernel(
    _ea_body,
    out_type=jax.ShapeDtypeStruct((NC, NPAD, D_EDGE), jnp.float32),
    mesh=_MESH,
    scratch_types=[
        pltpu.VMEM((EPT,), jnp.int32),
        pltpu.VMEM((NCHUNK, CH), jnp.int32),
        pltpu.VMEM((CH, D_EDGE), jnp.float32),
        pltpu.VMEM((CH, D), jnp.float32),
        pltpu.VMEM_SHARED((NPAD, D_EDGE), jnp.float32),
    ],
)


def _spmm_body(hs_hbm, src_hbm, dst3_hbm, out_hbm, src_v, dst_v, buf, agg_sh):
    core = lax.axis_index("c")
    w = _wid()
    base = w * EPT
    row0 = lax.axis_index("s") * RPT

    _zero_rows(buf, agg_sh, row0, D)
    _load_src_idx(src_hbm, src_v, base)
    _load_dst_idx(dst3_hbm, dst_v, w)
    plsc.subcore_barrier()

    def chunk(i, carry):
        pltpu.sync_copy(hs_hbm.at[src_v.at[pl.ds(i * CH, CH)]], buf)
        pltpu.sync_copy(buf, agg_sh.at[dst_v.at[i]], add=True)
        return carry

    lax.fori_loop(0, NCHUNK, chunk, 0)
    plsc.subcore_barrier()
    _copy_out(agg_sh, buf, out_hbm, core, row0, D)


_spmm_kernel = pl.kernel(
    _spmm_body,
    out_type=jax.ShapeDtypeStruct((NC, NPAD, D), jnp.float32),
    mesh=_MESH,
    scratch_types=[
        pltpu.VMEM((EPT,), jnp.int32),
        pltpu.VMEM((NCHUNK, CH), jnp.int32),
        pltpu.VMEM((CH, D), jnp.float32),
        pltpu.VMEM_SHARED((NPAD, D), jnp.float32),
    ],
)


RB = 1000


def _tc_b_body(x_ref, w0_ref, b0_ref, ws_ref, bs_ref, degp_ref, hs_ref, dis_ref):
    deg = degp_ref[0, :, 0] + degp_ref[1, :, 0] + 1.0
    dis = lax.rsqrt(deg)[:, None]
    h = jnp.maximum(
        jnp.dot(x_ref[...], w0_ref[...], preferred_element_type=jnp.float32)
        + b0_ref[...], 0.0)
    hl = jnp.dot(h, ws_ref[...], preferred_element_type=jnp.float32) + bs_ref[...]
    hs_ref[...] = hl * dis
    dis_ref[...] = jnp.broadcast_to(dis, (RB, D))


_tc_b = pl.pallas_call(
    _tc_b_body,
    grid=(N // RB,),
    in_specs=[
        pl.BlockSpec((RB, D), lambda i: (i, 0)),
        pl.BlockSpec((D, D), lambda i: (0, 0)),
        pl.BlockSpec((1, D), lambda i: (0, 0)),
        pl.BlockSpec((D, D), lambda i: (0, 0)),
        pl.BlockSpec((1, D), lambda i: (0, 0)),
        pl.BlockSpec((NC, RB, D_EDGE), lambda i: (0, i, 0)),
    ],
    out_specs=[
        pl.BlockSpec((RB, D), lambda i: (i, 0)),
        pl.BlockSpec((RB, D), lambda i: (i, 0)),
    ],
    out_shape=[
        jax.ShapeDtypeStruct((N, D), jnp.float32),
        jax.ShapeDtypeStruct((N, D), jnp.float32),
    ],
)


def _tc_e_body(p_ref, hs_ref, dis_ref, eap_ref, wes_ref, sg_ref, beta_ref,
               wn_ref, bn_ref, out_ref, *, last):
    s = p_ref[0] + p_ref[1]
    ea = eap_ref[0] + eap_ref[1]
    dis = dis_ref[...]
    agg = dis * (
        s + jnp.dot(ea, wes_ref[...], preferred_element_type=jnp.float32)
        + hs_ref[...])
    hbn = agg * sg_ref[...] + beta_ref[...]
    if last:
        out_ref[...] = hbn
    else:
        h = jnp.maximum(hbn, 0.0)
        out_ref[...] = (
            jnp.dot(h, wn_ref[...], preferred_element_type=jnp.float32)
            + bn_ref[...]) * dis


def _make_tc_e(last):
    return pl.pallas_call(
        functools.partial(_tc_e_body, last=last),
        grid=(N // RB,),
        in_specs=[
            pl.BlockSpec((NC, RB, D), lambda i: (0, i, 0)),
            pl.BlockSpec((RB, D), lambda i: (i, 0)),
            pl.BlockSpec((RB, D), lambda i: (i, 0)),
            pl.BlockSpec((NC, RB, D_EDGE), lambda i: (0, i, 0)),
            pl.BlockSpec((D_EDGE, D), lambda i: (0, 0)),
            pl.BlockSpec((1, D), lambda i: (0, 0)),
            pl.BlockSpec((1, D), lambda i: (0, 0)),
            pl.BlockSpec((D, D), lambda i: (0, 0)),
            pl.BlockSpec((1, D), lambda i: (0, 0)),
        ],
        out_specs=pl.BlockSpec((RB, D), lambda i: (i, 0)),
        out_shape=jax.ShapeDtypeStruct((N, D), jnp.float32),
    )


_tc_e_mid = _make_tc_e(last=False)
_tc_e_last = _make_tc_e(last=True)


def kernel(x, edge_index, edge_attr, W0, b0, Ws, bs, Wes, gammas, betas):
    ei = edge_index.astype(jnp.int32)
    src_e, dst_e = ei[0], ei[1]
    sg = (gammas / jnp.sqrt(1.0 + EPS)).astype(jnp.float32)

    dst3 = dst_e.reshape(NW, EPT // CH, CH)
    degp = _deg_kernel(dst3)
    hs, dis_col = _tc_b(
        x, W0, b0.reshape(1, D), Ws[0], bs[0].reshape(1, D), degp)
    eap = _ea_kernel(edge_attr, src_e, dst3, dis_col)

    for l in range(L):
        part = _spmm_kernel(hs, src_e, dst3)
        if l < L - 1:
            hs = _tc_e_mid(
                part, hs, dis_col, eap, Wes[l], sg[l].reshape(1, D),
                betas[l].reshape(1, D), Ws[l + 1], bs[l + 1].reshape(1, D))
        else:
            out = _tc_e_last(
                part, hs, dis_col, eap, Wes[l], sg[l].reshape(1, D),
                betas[l].reshape(1, D), Ws[l], bs[l].reshape(1, D))
    return out
